# baseline (device time: 38107 ns/iter reference)
import jax
import jax.numpy as jnp
from jax import lax
from jax.experimental import pallas as pl
from jax.experimental.pallas import tpu as pltpu

N_DEV = 8
N_ROUNDS = 3
N_LAYERS = 3
B, D = 64, 512
NC = 4
W = D // NC
N_SLOTS = N_LAYERS * NC * N_ROUNDS

MASKS = (1, 3, 4)


def kernel(x, Win0, Wout0, Win1, Wout1, Win2, Wout2):
    def body(
        x_ref,
        win0_ref,
        wout0_ref,
        win1_ref,
        wout1_ref,
        win2_ref,
        wout2_ref,
        out_ref,
        send_ref,
        recv_ref,
        tmp_ref,
        send_sems,
        recv_sems,
    ):
        my = lax.axis_index("i")
        wins = [win0_ref, win1_ref, win2_ref]
        wouts = [wout0_ref, wout1_ref, wout2_ref]

        barrier_sem = pltpu.get_barrier_semaphore()
        for m in MASKS:
            pl.semaphore_signal(
                barrier_sem,
                inc=1,
                device_id=(my ^ m,),
                device_id_type=pl.DeviceIdType.MESH,
            )
        pl.semaphore_wait(barrier_sem, len(MASKS))

        def slot(layer, c, r):
            return (layer * NC + c) * N_ROUNDS + r

        def start_rdma(layer, c, r, val):
            i = slot(layer, c, r)
            send_ref[i, :, :] = val
            rdma = pltpu.make_async_remote_copy(
                src_ref=send_ref.at[i],
                dst_ref=recv_ref.at[i],
                send_sem=send_sems.at[i],
                recv_sem=recv_sems.at[i],
                device_id=(my ^ MASKS[r],),
                device_id_type=pl.DeviceIdType.MESH,
            )
            rdma.start()
            return rdma

        bf16 = jnp.bfloat16
        h = jnp.maximum(
            jnp.dot(
                x_ref[:, :].astype(bf16),
                wins[0][:, :].astype(bf16),
                preferred_element_type=jnp.float32,
            ),
            0.0,
        ).astype(bf16)
        for layer in range(N_LAYERS):
            acc = [None] * NC
            rdmas = {}
            wout_b = wouts[layer][:, :].astype(bf16)
            for c in range(NC):
                acc[c] = jnp.dot(
                    h,
                    wout_b[:, c * W : (c + 1) * W],
                    preferred_element_type=jnp.float32,
                )
                rdmas[(c, 0)] = start_rdma(layer, c, 0, acc[c])
            hacc = None
            win_next = (
                wins[layer + 1][:, :].astype(bf16) if layer < N_LAYERS - 1 else None
            )
            for r in range(N_ROUNDS):
                for c in range(NC):
                    rdmas.pop((c, r)).wait()
                    acc[c] = acc[c] + recv_ref[slot(layer, c, r), :, :]
                    if r < N_ROUNDS - 1:
                        rdmas[(c, r + 1)] = start_rdma(layer, c, r + 1, acc[c])
                    elif layer < N_LAYERS - 1:
                        contrib = jnp.dot(
                            acc[c].astype(bf16),
                            win_next[c * W : (c + 1) * W, :],
                            preferred_element_type=jnp.float32,
                        )
                        hacc = contrib if hacc is None else hacc + contrib
                    else:
                        tmp_ref[:, :] = acc[c]
                        out_ref[:, c * W : (c + 1) * W] = tmp_ref[
                            pl.ds(my * (B // N_DEV), B // N_DEV), :
                        ]
            if layer < N_LAYERS - 1:
                h = jnp.maximum(hacc, 0.0).astype(bf16)

    return pl.pallas_call(
        body,
        out_shape=jax.ShapeDtypeStruct((B // N_DEV, D), jnp.float32),
        in_specs=[pl.BlockSpec(memory_space=pltpu.VMEM)] * 7,
        out_specs=pl.BlockSpec(memory_space=pltpu.VMEM),
        scratch_shapes=[
            pltpu.VMEM((N_SLOTS, B, W), jnp.float32),
            pltpu.VMEM((N_SLOTS, B, W), jnp.float32),
            pltpu.VMEM((B, W), jnp.float32),
            pltpu.SemaphoreType.DMA((N_SLOTS,)),
            pltpu.SemaphoreType.DMA((N_SLOTS,)),
        ],
        compiler_params=pltpu.CompilerParams(collective_id=0),
    )(x, Win0, Wout0, Win1, Wout1, Win2, Wout2)


# device time: 37482 ns/iter; 1.0167x vs baseline; 1.0167x over previous
import jax
import jax.numpy as jnp
from jax import lax
from jax.experimental import pallas as pl
from jax.experimental.pallas import tpu as pltpu

N_DEV = 8
N_LAYERS = 3
B, D = 64, 512
NC = 4
W = D // NC
N_LC = N_LAYERS * NC

A_MASKS = (1, 3, 2)
B_MASK = 4
ALL_MASKS = (1, 3, 2, 4)


def kernel(x, Win0, Wout0, Win1, Wout1, Win2, Wout2):
    def body(
        x_ref,
        win0_ref,
        wout0_ref,
        win1_ref,
        wout1_ref,
        win2_ref,
        wout2_ref,
        out_ref,
        send_ref,
        recv_ref,
        tmp_ref,
        send_sems,
        recv_sems,
    ):
        my = lax.axis_index("i")
        wins = [win0_ref, win1_ref, win2_ref]
        wouts = [wout0_ref, wout1_ref, wout2_ref]

        barrier_sem = pltpu.get_barrier_semaphore()
        for m in ALL_MASKS:
            pl.semaphore_signal(
                barrier_sem,
                inc=1,
                device_id=(my ^ m,),
                device_id_type=pl.DeviceIdType.MESH,
            )
        pl.semaphore_wait(barrier_sem, len(ALL_MASKS))

        def make_rdma(lc, phase_slot, mask, j):
            return pltpu.make_async_remote_copy(
                src_ref=send_ref.at[lc, phase_slot],
                dst_ref=recv_ref.at[lc, j],
                send_sem=send_sems.at[lc, j],
                recv_sem=recv_sems.at[lc, j],
                device_id=(my ^ mask,),
                device_id_type=pl.DeviceIdType.MESH,
            )

        def start_a(lc, val):
            send_ref[lc, 0, :, :] = val
            rdmas = [make_rdma(lc, 0, m, j) for j, m in enumerate(A_MASKS)]
            for r in rdmas:
                r.start()
            return rdmas

        def start_b(lc, val):
            send_ref[lc, 1, :, :] = val
            r = make_rdma(lc, 1, B_MASK, 3)
            r.start()
            return [r]

        h = jnp.maximum(
            jnp.dot(x_ref[:, :], wins[0][:, :], preferred_element_type=jnp.float32),
            0.0,
        )
        for layer in range(N_LAYERS):
            acc = [None] * NC
            rdmas = {}
            for c in range(NC):
                lc = layer * NC + c
                acc[c] = jnp.dot(
                    h,
                    wouts[layer][:, c * W : (c + 1) * W],
                    preferred_element_type=jnp.float32,
                )
                rdmas[c] = start_a(lc, acc[c])
            hacc = None
            for c in range(NC):
                lc = layer * NC + c
                for r in rdmas[c]:
                    r.wait()
                acc[c] = (
                    acc[c]
                    + recv_ref[lc, 0, :, :]
                    + recv_ref[lc, 1, :, :]
                    + recv_ref[lc, 2, :, :]
                )
                rdmas[c] = start_b(lc, acc[c])
            for c in range(NC):
                lc = layer * NC + c
                for r in rdmas[c]:
                    r.wait()
                acc[c] = acc[c] + recv_ref[lc, 3, :, :]
                if layer < N_LAYERS - 1:
                    contrib = jnp.dot(
                        acc[c],
                        wins[layer + 1][c * W : (c + 1) * W, :],
                        preferred_element_type=jnp.float32,
                    )
                    hacc = contrib if hacc is None else hacc + contrib
                else:
                    tmp_ref[:, :] = acc[c]
                    out_ref[:, c * W : (c + 1) * W] = tmp_ref[
                        pl.ds(my * (B // N_DEV), B // N_DEV), :
                    ]
            if layer < N_LAYERS - 1:
                h = jnp.maximum(hacc, 0.0)

    return pl.pallas_call(
        body,
        out_shape=jax.ShapeDtypeStruct((B // N_DEV, D), jnp.float32),
        in_specs=[pl.BlockSpec(memory_space=pltpu.VMEM)] * 7,
        out_specs=pl.BlockSpec(memory_space=pltpu.VMEM),
        scratch_shapes=[
            pltpu.VMEM((N_LC, 2, B, W), jnp.float32),
            pltpu.VMEM((N_LC, 4, B, W), jnp.float32),
            pltpu.VMEM((B, W), jnp.float32),
            pltpu.SemaphoreType.DMA((N_LC, 4)),
            pltpu.SemaphoreType.DMA((N_LC, 4)),
        ],
        compiler_params=pltpu.CompilerParams(collective_id=0),
    )(x, Win0, Wout0, Win1, Wout1, Win2, Wout2)
